# baseline (speedup 1.0000x reference)
import jax
import jax.numpy as jnp
from jax import lax
from jax.experimental import pallas as pl
from jax.experimental.pallas import tpu as pltpu
from jax.experimental.pallas import tpu_sc as plsc

_NC = 2
_NS = 16
_NW = _NC * _NS
_CH = 128


def _mesh():
    return plsc.VectorSubcoreMesh(core_axis_name="c", subcore_axis_name="s",
                                  num_cores=_NC, num_subcores=_NS)


def _sc_degree(dst_p, n_rows, u0, u1):
    def body(dst_hbm, out_hbm, dst_v, hist_v):
        c = lax.axis_index("c")
        s = lax.axis_index("s")
        w = s * _NC + c
        z16 = jnp.zeros((16,), jnp.float32)

        def zrow(r, carry):
            hist_v[pl.ds(r * 16, 16)] = z16
            return carry

        lax.fori_loop(0, n_rows // 16, zrow, 0)

        base = jnp.where(c == 0, s * u0, _NS * u0 + s * u1)
        nst = jnp.where(c == 0, u0 // 16, u1 // 16)
        pltpu.sync_copy(dst_hbm.at[pl.ds(base, u0)], dst_v)
        ones = jnp.ones((16,), jnp.float32)

        def it(i, carry):
            idx = dst_v[pl.ds(i * 16, 16)]
            plsc.addupdate_scatter(hist_v, [idx], ones)
            return carry

        lax.fori_loop(0, nst, it, 0)
        pltpu.sync_copy(hist_v, out_hbm.at[w])

    f = pl.kernel(
        body,
        out_type=jax.ShapeDtypeStruct((_NW, n_rows), jnp.float32),
        mesh=_mesh(),
        compiler_params=pltpu.CompilerParams(needs_layout_passes=False),
        scratch_types=[
            pltpu.VMEM((u0,), jnp.int32),
            pltpu.VMEM((n_rows,), jnp.float32),
        ],
    )
    return f(dst_p)


_NBUF = 2


def _sc_scatter_rows(vals, src_p_flat, dst_p_flat):
    n_rows, f = vals.shape
    steps = src_p_flat.shape[0] // (_NW * _CH)
    rpt = n_rows // _NS
    n16 = f // 16

    hch = _CH // 2

    def body(vals_hbm, src_hbm, dst_hbm, out_hbm, s0, d0, b0,
             acc_sh, gsem, gsem1):
        c = lax.axis_index("c")
        s = lax.axis_index("s")
        w = s * _NC + c
        z16 = jnp.zeros((16,), jnp.float32)

        def zrow(r, carry):
            for kk in range(n16):
                b0[r, pl.ds(kk * 16, 16)] = z16
            return carry

        lax.fori_loop(0, _CH, zrow, 0)

        row0 = s * rpt
        for t in range(rpt // _CH):
            pltpu.sync_copy(b0, acc_sh.at[pl.ds(row0 + t * _CH, _CH)])
        plsc.subcore_barrier()

        n0 = (2 * steps * 66) // 100
        n1 = 2 * steps - n0
        steps_w = jnp.where(c == 0, n0, n1)
        base = jnp.where(c == 0, s * n0, _NS * n0 + s * n1) * _CH

        def it(jj, carry):
            off = base + jj * _CH
            pltpu.sync_copy(src_hbm.at[pl.ds(off, _CH)], s0)
            pltpu.sync_copy(dst_hbm.at[pl.ds(off, _CH)], d0)
            pltpu.async_copy(vals_hbm.at[s0], b0, gsem).wait()
            pltpu.sync_copy(b0, acc_sh.at[d0], add=True)
            return carry

        lax.fori_loop(0, steps_w, it, 0)
        plsc.subcore_barrier()
        pltpu.sync_copy(acc_sh.at[pl.ds(row0, rpt)],
                        out_hbm.at[c, pl.ds(row0, rpt)])

    f_k = pl.kernel(
        body,
        out_type=jax.ShapeDtypeStruct((_NC, n_rows, f), jnp.float32),
        mesh=_mesh(),
        compiler_params=pltpu.CompilerParams(needs_layout_passes=False),
        scratch_types=[
            pltpu.VMEM((_CH,), jnp.int32),
            pltpu.VMEM((_CH,), jnp.int32),
            pltpu.VMEM((_CH, f), jnp.float32),
            pltpu.VMEM_SHARED((n_rows, f), jnp.float32),
            pltpu.SemaphoreType.DMA,
            pltpu.SemaphoreType.DMA,
        ],
    )
    return f_k(vals, src_p_flat, dst_p_flat)


def _sc_scatter_cols(g2, src_p, dst_p, u0, u1):
    _, n_rows = g2.shape

    def body(g_hbm, src_hbm, dst_hbm, out_hbm, g0_v, g1_v, a0_v, a1_v,
             src_v, dst_v):
        c = lax.axis_index("c")
        s = lax.axis_index("s")
        w = s * _NC + c
        z16 = jnp.zeros((16,), jnp.float32)

        def zrow(r, carry):
            a0_v[pl.ds(r * 16, 16)] = z16
            a1_v[pl.ds(r * 16, 16)] = z16
            return carry

        lax.fori_loop(0, n_rows // 16, zrow, 0)

        base = jnp.where(c == 0, s * u0, _NS * u0 + s * u1)
        nst = jnp.where(c == 0, u0 // 16, u1 // 16)
        pltpu.sync_copy(g_hbm.at[0], g0_v)
        pltpu.sync_copy(g_hbm.at[1], g1_v)
        pltpu.sync_copy(src_hbm.at[pl.ds(base, u0)], src_v)
        pltpu.sync_copy(dst_hbm.at[pl.ds(base, u0)], dst_v)

        def it(i, carry):
            sidx = src_v[pl.ds(i * 16, 16)]
            didx = dst_v[pl.ds(i * 16, 16)]
            v0 = plsc.load_gather(g0_v, [sidx])
            v1 = plsc.load_gather(g1_v, [sidx])
            plsc.addupdate_scatter(a0_v, [didx], v0)
            plsc.addupdate_scatter(a1_v, [didx], v1)
            return carry

        lax.fori_loop(0, nst, it, 0)
        pltpu.sync_copy(a0_v, out_hbm.at[w, 0])
        pltpu.sync_copy(a1_v, out_hbm.at[w, 1])

    f_k = pl.kernel(
        body,
        out_type=jax.ShapeDtypeStruct((_NW, 2, n_rows), jnp.float32),
        mesh=_mesh(),
        compiler_params=pltpu.CompilerParams(needs_layout_passes=False),
        scratch_types=[
            pltpu.VMEM((n_rows,), jnp.float32),
            pltpu.VMEM((n_rows,), jnp.float32),
            pltpu.VMEM((n_rows,), jnp.float32),
            pltpu.VMEM((n_rows,), jnp.float32),
            pltpu.VMEM((u0,), jnp.int32),
            pltpu.VMEM((u0,), jnp.int32),
        ],
    )
    return f_k(g2, src_p, dst_p)


def _tc_deg_to_dinv(hist):
    _, hr, _ = hist.shape

    def body(h_ref, o_ref):
        deg = jnp.sum(h_ref[...], axis=0) + 1.0
        o_ref[...] = lax.rsqrt(deg)

    return pl.pallas_call(
        body, out_shape=jax.ShapeDtypeStruct((hr, 128), jnp.float32))(hist)


def _tc_scale(xp, d_col):
    nr, f = xp.shape
    blk = 512

    def body(x_ref, d_ref, o_ref):
        o_ref[...] = x_ref[...] * d_ref[...]

    return pl.pallas_call(
        body,
        grid=(nr // blk,),
        in_specs=[pl.BlockSpec((blk, f), lambda i: (i, 0)),
                  pl.BlockSpec((blk, 1), lambda i: (i, 0))],
        out_specs=pl.BlockSpec((blk, f), lambda i: (i, 0)),
        out_shape=jax.ShapeDtypeStruct((nr, f), jnp.float32))(xp, d_col)


def _tc_mlp(a0, a1, xs, d_col, W1, b1r, W2p):
    nr, f = xs.shape
    h = W1.shape[1]
    f2 = W2p.shape[1]
    blk = 512

    def body(a0_ref, a1_ref, xs_ref, d_ref, w1_ref, b1_ref, w2_ref, o_ref):
        p1 = (a0_ref[...] + a1_ref[...] + xs_ref[...]) * d_ref[...]
        hh = jnp.dot(p1, w1_ref[...], preferred_element_type=jnp.float32)
        hh = jnp.maximum(hh + b1_ref[...], 0.0)
        g = jnp.dot(hh, w2_ref[...], preferred_element_type=jnp.float32)
        o_ref[...] = g * d_ref[...]

    return pl.pallas_call(
        body,
        grid=(nr // blk,),
        in_specs=[pl.BlockSpec((blk, f), lambda i: (i, 0)),
                  pl.BlockSpec((blk, f), lambda i: (i, 0)),
                  pl.BlockSpec((blk, f), lambda i: (i, 0)),
                  pl.BlockSpec((blk, 1), lambda i: (i, 0)),
                  pl.BlockSpec((f, h), lambda i: (0, 0)),
                  pl.BlockSpec((1, h), lambda i: (0, 0)),
                  pl.BlockSpec((h, f2), lambda i: (0, 0))],
        out_specs=pl.BlockSpec((blk, f2), lambda i: (i, 0)),
        out_shape=jax.ShapeDtypeStruct((nr, f2), jnp.float32))(
            a0, a1, xs, d_col, W1, b1r, W2p)


def _tc_final(agg2s, gstack, dinv, b2s):
    _, hr2, _ = agg2s.shape
    hr = hr2 // 2

    def body(a_ref, g_ref, d_ref, b2_ref, o_ref):
        a = jnp.sum(a_ref[...], axis=0) + g_ref[...]
        d = d_ref[...]
        p0 = a[0:hr, :] * d + b2_ref[0, 0]
        p1 = a[hr:hr2, :] * d + b2_ref[0, 1]
        m = jnp.maximum(p0, p1)
        lse = m + jnp.log(jnp.exp(p0 - m) + jnp.exp(p1 - m))
        o_ref[0] = p0 - lse
        o_ref[1] = p1 - lse

    return pl.pallas_call(
        body,
        in_specs=[pl.BlockSpec(),
                  pl.BlockSpec(),
                  pl.BlockSpec(),
                  pl.BlockSpec(memory_space=pltpu.SMEM)],
        out_shape=jax.ShapeDtypeStruct((2, hr, 128), jnp.float32))(
            agg2s, gstack, dinv, b2s)


def kernel(x, edge_index, W1, b1, W2, b2):
    n, f = x.shape
    h = W1.shape[1]
    c_out = W2.shape[1]
    e = edge_index.shape[1]

    nr = ((n + 2047) // 2048) * 2048
    grp = _NW * _CH
    e_pad = ((e + grp - 1) // grp) * grp
    f2 = 16
    e_w = e_pad // _NW
    u0 = ((2 * e_w * 66) // 100) // 16 * 16
    u1 = 2 * e_w - u0
    e_pad2 = e_pad + (u0 - u1)

    src_p = jnp.concatenate(
        [edge_index[0].astype(jnp.int32),
         jnp.full((e_pad2 - e,), n, jnp.int32)])
    dst_p = jnp.concatenate(
        [edge_index[1].astype(jnp.int32),
         jnp.full((e_pad2 - e,), n, jnp.int32)])
    xp = jnp.pad(x, ((0, nr - n), (0, 0)))
    W2p = jnp.pad(W2, ((0, 0), (0, f2 - c_out)))
    b1r = b1.reshape(1, h)
    b2s = b2.reshape(1, c_out)

    hist = _sc_degree(dst_p, nr, u0, u1).reshape(_NW, nr // 128, 128)
    dinv = _tc_deg_to_dinv(hist)
    d_col = dinv.reshape(nr, 1)
    xs = _tc_scale(xp, d_col)
    agg1 = _sc_scatter_rows(xs, src_p, dst_p)
    gs = _tc_mlp(agg1[0], agg1[1], xs, d_col, W1, b1r, W2p)
    g2 = jnp.transpose(gs[:, :c_out])
    agg2 = _sc_scatter_cols(g2, src_p, dst_p, u0, u1)
    outp = _tc_final(agg2.reshape(_NW, 2 * nr // 128, 128),
                     g2.reshape(2 * nr // 128, 128), dinv, b2s)
    return outp.reshape(2, nr)[:, :n].T


# speedup vs baseline: 1.5251x; 1.5251x over previous
import jax
import jax.numpy as jnp
from jax import lax
from jax.experimental import pallas as pl
from jax.experimental.pallas import tpu as pltpu
from jax.experimental.pallas import tpu_sc as plsc

_NC = 2
_NS = 16
_NW = _NC * _NS
_CH = 128


def _mesh():
    return plsc.VectorSubcoreMesh(core_axis_name="c", subcore_axis_name="s",
                                  num_cores=_NC, num_subcores=_NS)


def _sc_degree(dst_p, n_rows, u0, u1):
    def body(dst_hbm, out_hbm, dst_v, hist_v):
        c = lax.axis_index("c")
        s = lax.axis_index("s")
        w = s * _NC + c
        z16 = jnp.zeros((16,), jnp.float32)

        def zrow(r, carry):
            hist_v[pl.ds(r * 16, 16)] = z16
            return carry

        lax.fori_loop(0, n_rows // 16, zrow, 0)

        base = jnp.where(c == 0, s * u0, _NS * u0 + s * u1)
        nst = jnp.where(c == 0, u0 // 16, u1 // 16)
        pltpu.sync_copy(dst_hbm.at[pl.ds(base, u0)], dst_v)
        ones = jnp.ones((16,), jnp.float32)

        def it(i, carry):
            idx = dst_v[pl.ds(i * 16, 16)]
            plsc.addupdate_scatter(hist_v, [idx], ones)
            return carry

        lax.fori_loop(0, nst, it, 0)
        pltpu.sync_copy(hist_v, out_hbm.at[w])

    f = pl.kernel(
        body,
        out_type=jax.ShapeDtypeStruct((_NW, n_rows), jnp.float32),
        mesh=_mesh(),
        compiler_params=pltpu.CompilerParams(needs_layout_passes=False),
        scratch_types=[
            pltpu.VMEM((u0,), jnp.int32),
            pltpu.VMEM((n_rows,), jnp.float32),
        ],
    )
    return f(dst_p)


_NBUF = 2


def _sc_scatter_rows(vals, src_p_flat, dst_p_flat):
    n_rows, f = vals.shape
    steps = src_p_flat.shape[0] // (_NW * _CH)
    rpt = n_rows // _NS
    n16 = f // 16

    hch = _CH // 2

    def body(vals_hbm, src_hbm, dst_hbm, out_hbm, s0, d0, b0,
             acc_sh, gsem, gsem1):
        c = lax.axis_index("c")
        s = lax.axis_index("s")
        w = s * _NC + c
        z16 = jnp.zeros((16,), jnp.float32)

        def zrow(r, carry):
            for kk in range(n16):
                b0[r, pl.ds(kk * 16, 16)] = z16
            return carry

        lax.fori_loop(0, _CH, zrow, 0)

        row0 = s * rpt
        for t in range(rpt // _CH):
            pltpu.sync_copy(b0, acc_sh.at[pl.ds(row0 + t * _CH, _CH)])
        plsc.subcore_barrier()

        n0 = (2 * steps * 66) // 100
        n1 = 2 * steps - n0
        steps_w = jnp.where(c == 0, n0, n1)
        base = jnp.where(c == 0, s * n0, _NS * n0 + s * n1) * _CH

        def it(jj, carry):
            off = base + jj * _CH
            pltpu.sync_copy(src_hbm.at[pl.ds(off, _CH)], s0)
            pltpu.sync_copy(dst_hbm.at[pl.ds(off, _CH)], d0)
            pltpu.async_copy(vals_hbm.at[s0], b0, gsem).wait()
            pltpu.sync_copy(b0, acc_sh.at[d0], add=True)
            return carry

        lax.fori_loop(0, steps_w, it, 0)
        plsc.subcore_barrier()
        pltpu.sync_copy(acc_sh.at[pl.ds(row0, rpt)],
                        out_hbm.at[c, pl.ds(row0, rpt)])

    f_k = pl.kernel(
        body,
        out_type=jax.ShapeDtypeStruct((_NC, n_rows, f), jnp.float32),
        mesh=_mesh(),
        compiler_params=pltpu.CompilerParams(needs_layout_passes=False),
        scratch_types=[
            pltpu.VMEM((_CH,), jnp.int32),
            pltpu.VMEM((_CH,), jnp.int32),
            pltpu.VMEM((_CH, f), jnp.float32),
            pltpu.VMEM_SHARED((n_rows, f), jnp.float32),
            pltpu.SemaphoreType.DMA,
            pltpu.SemaphoreType.DMA,
        ],
    )
    return f_k(vals, src_p_flat, dst_p_flat)


def _sc_scatter_cols(g2, src_p, dst_p, u0, u1):
    _, n_rows = g2.shape

    def body(g_hbm, src_hbm, dst_hbm, out_hbm, g0_v, g1_v, a0_v, a1_v,
             src_v, dst_v):
        c = lax.axis_index("c")
        s = lax.axis_index("s")
        w = s * _NC + c
        z16 = jnp.zeros((16,), jnp.float32)

        def zrow(r, carry):
            a0_v[pl.ds(r * 16, 16)] = z16
            a1_v[pl.ds(r * 16, 16)] = z16
            return carry

        lax.fori_loop(0, n_rows // 16, zrow, 0)

        base = jnp.where(c == 0, s * u0, _NS * u0 + s * u1)
        nst = jnp.where(c == 0, u0 // 16, u1 // 16)
        pltpu.sync_copy(g_hbm.at[0], g0_v)
        pltpu.sync_copy(g_hbm.at[1], g1_v)
        pltpu.sync_copy(src_hbm.at[pl.ds(base, u0)], src_v)
        pltpu.sync_copy(dst_hbm.at[pl.ds(base, u0)], dst_v)

        def it(i, carry):
            sidx = src_v[pl.ds(i * 16, 16)]
            didx = dst_v[pl.ds(i * 16, 16)]
            v0 = plsc.load_gather(g0_v, [sidx])
            v1 = plsc.load_gather(g1_v, [sidx])
            plsc.addupdate_scatter(a0_v, [didx], v0)
            plsc.addupdate_scatter(a1_v, [didx], v1)
            return carry

        lax.fori_loop(0, nst, it, 0)
        pltpu.sync_copy(a0_v, out_hbm.at[w, 0])
        pltpu.sync_copy(a1_v, out_hbm.at[w, 1])

    f_k = pl.kernel(
        body,
        out_type=jax.ShapeDtypeStruct((_NW, 2, n_rows), jnp.float32),
        mesh=_mesh(),
        compiler_params=pltpu.CompilerParams(needs_layout_passes=False),
        scratch_types=[
            pltpu.VMEM((n_rows,), jnp.float32),
            pltpu.VMEM((n_rows,), jnp.float32),
            pltpu.VMEM((n_rows,), jnp.float32),
            pltpu.VMEM((n_rows,), jnp.float32),
            pltpu.VMEM((u0,), jnp.int32),
            pltpu.VMEM((u0,), jnp.int32),
        ],
    )
    return f_k(g2, src_p, dst_p)


def _tc_deg_to_dinv(hist):
    _, hr, _ = hist.shape

    def body(h_ref, o_ref):
        deg = jnp.sum(h_ref[...], axis=0) + 1.0
        o_ref[...] = lax.rsqrt(deg)

    return pl.pallas_call(
        body, out_shape=jax.ShapeDtypeStruct((hr, 128), jnp.float32))(hist)


def _tc_scale(xp, d_col):
    nr, f = xp.shape
    blk = 512

    def body(x_ref, d_ref, o_ref):
        o_ref[...] = x_ref[...] * d_ref[...]

    return pl.pallas_call(
        body,
        grid=(nr // blk,),
        in_specs=[pl.BlockSpec((blk, f), lambda i: (i, 0)),
                  pl.BlockSpec((blk, 1), lambda i: (i, 0))],
        out_specs=pl.BlockSpec((blk, f), lambda i: (i, 0)),
        out_shape=jax.ShapeDtypeStruct((nr, f), jnp.float32))(xp, d_col)


def _tc_mlp(a0, a1, xs, d_col, W1, b1r, W2p):
    nr, f = xs.shape
    h = W1.shape[1]
    f2 = W2p.shape[1]
    blk = 512

    def body(a0_ref, a1_ref, xs_ref, d_ref, w1_ref, b1_ref, w2_ref, o_ref):
        p1 = (a0_ref[...] + a1_ref[...] + xs_ref[...]) * d_ref[...]
        hh = jnp.dot(p1, w1_ref[...], preferred_element_type=jnp.float32)
        hh = jnp.maximum(hh + b1_ref[...], 0.0)
        g = jnp.dot(hh, w2_ref[...], preferred_element_type=jnp.float32)
        o_ref[...] = g * d_ref[...]

    return pl.pallas_call(
        body,
        grid=(nr // blk,),
        in_specs=[pl.BlockSpec((blk, f), lambda i: (i, 0)),
                  pl.BlockSpec((blk, f), lambda i: (i, 0)),
                  pl.BlockSpec((blk, f), lambda i: (i, 0)),
                  pl.BlockSpec((blk, 1), lambda i: (i, 0)),
                  pl.BlockSpec((f, h), lambda i: (0, 0)),
                  pl.BlockSpec((1, h), lambda i: (0, 0)),
                  pl.BlockSpec((h, f2), lambda i: (0, 0))],
        out_specs=pl.BlockSpec((blk, f2), lambda i: (i, 0)),
        out_shape=jax.ShapeDtypeStruct((nr, f2), jnp.float32))(
            a0, a1, xs, d_col, W1, b1r, W2p)


def _tc_final(agg2s, gstack, dinv, b2s):
    _, hr2, _ = agg2s.shape
    hr = hr2 // 2

    def body(a_ref, g_ref, d_ref, b2_ref, o_ref):
        a = jnp.sum(a_ref[...], axis=0) + g_ref[...]
        d = d_ref[...]
        p0 = a[0:hr, :] * d + b2_ref[0, 0]
        p1 = a[hr:hr2, :] * d + b2_ref[0, 1]
        m = jnp.maximum(p0, p1)
        lse = m + jnp.log(jnp.exp(p0 - m) + jnp.exp(p1 - m))
        o_ref[0] = p0 - lse
        o_ref[1] = p1 - lse

    return pl.pallas_call(
        body,
        in_specs=[pl.BlockSpec(),
                  pl.BlockSpec(),
                  pl.BlockSpec(),
                  pl.BlockSpec(memory_space=pltpu.SMEM)],
        out_shape=jax.ShapeDtypeStruct((2, hr, 128), jnp.float32))(
            agg2s, gstack, dinv, b2s)


def kernel(x, edge_index, W1, b1, W2, b2):
    n, f = x.shape
    h = W1.shape[1]
    c_out = W2.shape[1]
    e = edge_index.shape[1]

    nr = ((n + 2047) // 2048) * 2048
    grp = _NW * _CH
    e_pad = ((e + grp - 1) // grp) * grp
    f2 = 16
    e_w = e_pad // _NW
    u0 = ((2 * e_w * 66) // 100) // 16 * 16
    u1 = 2 * e_w - u0
    e_pad2 = e_pad + (u0 - u1)

    pad_idx = n + jnp.arange(e_pad2 - e, dtype=jnp.int32) % (nr - n)
    src_p = jnp.concatenate([edge_index[0].astype(jnp.int32), pad_idx])
    dst_p = jnp.concatenate([edge_index[1].astype(jnp.int32), pad_idx])
    xp = jnp.pad(x, ((0, nr - n), (0, 0)))
    W2p = jnp.pad(W2, ((0, 0), (0, f2 - c_out)))
    b1r = b1.reshape(1, h)
    b2s = b2.reshape(1, c_out)

    hist = _sc_degree(dst_p, nr, u0, u1).reshape(_NW, nr // 128, 128)
    dinv = _tc_deg_to_dinv(hist)
    d_col = dinv.reshape(nr, 1)
    xs = _tc_scale(xp, d_col)
    agg1 = _sc_scatter_rows(xs, src_p, dst_p)
    gs = _tc_mlp(agg1[0], agg1[1], xs, d_col, W1, b1r, W2p)
    g2 = jnp.transpose(gs[:, :c_out])
    agg2 = _sc_scatter_cols(g2, src_p, dst_p, u0, u1)
    outp = _tc_final(agg2.reshape(_NW, 2 * nr // 128, 128),
                     g2.reshape(2 * nr // 128, 128), dinv, b2s)
    return outp.reshape(2, nr)[:, :n].T
